# baseline (speedup 1.0000x reference)
import jax
import jax.numpy as jnp
from jax import lax
from jax.experimental import pallas as pl
from jax.experimental.pallas import tpu as pltpu

_G = 128
_R = 8
_NBUF = 4


def _tc_loss_fn(N, C):
    CH = _R * _G
    steps = N // CH

    def start(x_hbm, s_hbm, xbuf, sbuf, xsem, ssem, chunk, slot):
        pltpu.make_async_copy(
            x_hbm.at[pl.ds(chunk * CH, CH)], xbuf.at[slot],
            xsem.at[slot]).start()
        pltpu.make_async_copy(
            s_hbm.at[pl.ds(chunk * CH, CH)], sbuf.at[slot],
            ssem.at[slot]).start()

    def body(x_hbm, s_hbm, t_ref, out_ref, xbuf, sbuf, xsem, ssem):
        i = pl.program_id(0)

        @pl.when(i == 0)
        def _prologue():
            out_ref[0, 0] = 0.0
            for b in range(_NBUF):
                start(x_hbm, s_hbm, xbuf, sbuf, xsem, ssem, b, b)

        slot = lax.rem(i, _NBUF)
        pltpu.make_async_copy(x_hbm.at[pl.ds(i * CH, CH)], xbuf.at[slot],
                              xsem.at[slot]).wait()
        pltpu.make_async_copy(s_hbm.at[pl.ds(i * CH, CH)], sbuf.at[slot],
                              ssem.at[slot]).wait()

        x = xbuf[slot].reshape(_R, _G, C)
        sv = sbuf[slot].reshape(_R, _G, C)
        cols = lax.broadcasted_iota(jnp.int32, (_R, _G, C), 2)
        oh = cols == t_ref[...][:, :, None]
        m = jnp.max(x, axis=2)
        s = jnp.sum(jnp.exp(x - m[:, :, None]), axis=2)
        lse = m + jnp.log(s)
        logit_t = jnp.sum(jnp.where(oh, x, 0.0), axis=2)
        sim_t = jnp.sum(jnp.where(oh, sv, 0.0), axis=2)
        w = 10.0 / (1.0 + jnp.exp(4.0 * sim_t))
        out_ref[0, 0] += jnp.sum(w * (lse - logit_t))

        @pl.when(i + _NBUF < steps)
        def _prefetch():
            start(x_hbm, s_hbm, xbuf, sbuf, xsem, ssem, i + _NBUF, slot)

    return pl.pallas_call(
        body,
        grid=(steps,),
        in_specs=[
            pl.BlockSpec(memory_space=pltpu.MemorySpace.HBM),
            pl.BlockSpec(memory_space=pltpu.MemorySpace.HBM),
            pl.BlockSpec((_R, _G), lambda i: (i, 0)),
        ],
        out_specs=pl.BlockSpec(memory_space=pltpu.MemorySpace.SMEM),
        out_shape=jax.ShapeDtypeStruct((1, 1), jnp.float32),
        scratch_shapes=[
            pltpu.VMEM((_NBUF, CH, C), jnp.float32),
            pltpu.VMEM((_NBUF, CH, C), jnp.float32),
            pltpu.SemaphoreType.DMA((_NBUF,)),
            pltpu.SemaphoreType.DMA((_NBUF,)),
        ],
        compiler_params=pltpu.CompilerParams(
            dimension_semantics=("arbitrary",)),
    )


def kernel(src_logits, hoi_text_similarity, target_classes_i):
    N, C = src_logits.shape
    t2 = target_classes_i.astype(jnp.int32).reshape(N // _G, _G)
    out = _tc_loss_fn(N, C)(src_logits, hoi_text_similarity, t2)
    return out[0, 0] / N
